# baseline (speedup 1.0000x reference)
import functools

import jax
import jax.numpy as jnp
from jax import lax
from jax.experimental import pallas as pl
from jax.experimental.pallas import tpu as pltpu
from jax.experimental.pallas import tpu_sc as plsc

_LANES = 16


def _sc_degree(dst_p, n_nodes, NC, NS):
    _, C, _, K = dst_p.shape
    mesh = plsc.VectorSubcoreMesh(core_axis_name="c", subcore_axis_name="s")

    @functools.partial(
        pl.kernel,
        out_type=jax.ShapeDtypeStruct((NC, 1, n_nodes), jnp.float32),
        mesh=mesh,
        scratch_types=[
            pltpu.VMEM((C, 1, K), jnp.int32),
            pltpu.VMEM((K,), jnp.float32),
            pltpu.VMEM((K,), jnp.int32),
            pltpu.VMEM((n_nodes,), jnp.float32),
            pltpu.VMEM_SHARED((n_nodes,), jnp.float32),
        ],
    )
    def deg_kernel(dst_hbm, out_hbm, dst_v, w_v, idx_v, zero_v, deg_sh):
        cid = lax.axis_index("c")
        sid = lax.axis_index("s")

        def fill_w(i, _):
            w_v[pl.ds(i * _LANES, _LANES)] = jnp.full(
                (_LANES,), 0.5, jnp.float32)
            return 0

        lax.fori_loop(0, K // _LANES, fill_w, 0)

        @pl.when(sid == 0)
        def _():
            def z(i, _):
                zero_v[pl.ds(i * _LANES, _LANES)] = jnp.zeros(
                    (_LANES,), jnp.float32)
                return 0

            lax.fori_loop(0, n_nodes // _LANES, z, 0)
            pltpu.sync_copy(zero_v, deg_sh)

        pltpu.sync_copy(dst_hbm.at[sid], dst_v)
        plsc.subcore_barrier()

        def body(j, _):
            for g in range(K // _LANES):
                idx_v[pl.ds(g * _LANES, _LANES)] = (
                    dst_v[j, 0, pl.ds(g * _LANES, _LANES)])
            pltpu.sync_copy(w_v, deg_sh.at[idx_v], add=True)
            return 0

        lax.fori_loop(0, C, body, 0)
        plsc.subcore_barrier()

        @pl.when(sid == 0)
        def _():
            pltpu.sync_copy(deg_sh, out_hbm.at[cid, 0])

    return deg_kernel(dst_p)


def _sc_message(g0, src_p, dst_p, NC, NS):
    N, H = g0.shape
    _, C, _, K = src_p.shape
    HN = N // NC
    NTRASH = 16
    RPT = 8 * (HN // (8 * NS))
    REM = HN - NS * RPT
    ZR = 24
    KG = K // _LANES
    mesh = plsc.VectorSubcoreMesh(core_axis_name="c", subcore_axis_name="s")

    NBUF = 5

    @functools.partial(
        pl.kernel,
        out_type=jax.ShapeDtypeStruct((NC, HN, H), jnp.float32),
        mesh=mesh,
        scratch_types=(
            [pltpu.VMEM((K, H), jnp.float32) for _ in range(NBUF)]
            + [pltpu.VMEM((K,), jnp.int32) for _ in range(NBUF)]
            + [pltpu.VMEM((K,), jnp.int32) for _ in range(NBUF)]
            + [pltpu.VMEM((K,), jnp.int32) for _ in range(NBUF)]
            + [pltpu.VMEM((ZR, H), jnp.float32)]
            + [pltpu.VMEM_SHARED((HN + NTRASH, H), jnp.float32)]
            + [pltpu.SemaphoreType.DMA for _ in range(4 * NBUF)]
        ),
    )
    def msg_kernel(g0_hbm, src_hbm, dst_hbm, out_hbm, *sc):
        bufs = sc[0:NBUF]
        sbufs = sc[NBUF:2 * NBUF]
        dbufs = sc[2 * NBUF:3 * NBUF]
        dls = sc[3 * NBUF:4 * NBUF]
        zbuf = sc[4 * NBUF]
        acc_sh = sc[4 * NBUF + 1]
        iss = sc[4 * NBUF + 2:5 * NBUF + 2]
        jss = sc[5 * NBUF + 2:6 * NBUF + 2]
        gss = sc[6 * NBUF + 2:7 * NBUF + 2]
        sss = sc[7 * NBUF + 2:8 * NBUF + 2]
        cid = lax.axis_index("c")
        sid = lax.axis_index("s")

        nvec = H // _LANES

        def z(i, _):
            zbuf[i // nvec, pl.ds((i % nvec) * _LANES, _LANES)] = (
                jnp.zeros((_LANES,), jnp.float32))
            return 0

        lax.fori_loop(0, ZR * nvec, z, 0)
        for p in range(RPT // ZR):
            pltpu.sync_copy(zbuf, acc_sh.at[pl.ds(sid * RPT + p * ZR, ZR)])

        @pl.when(sid == 0)
        def _():
            pltpu.sync_copy(zbuf, acc_sh.at[pl.ds(NS * RPT, REM + NTRASH)])

        plsc.subcore_barrier()

        lo = cid * HN
        trash = HN + lax.iota(jnp.int32, _LANES)

        def remap(dbuf, dl):
            for g in range(KG):
                d = dbuf[pl.ds(g * _LANES, _LANES)] - lo
                ok = (d >= 0) & (d < HN)
                dl[pl.ds(g * _LANES, _LANES)] = jnp.where(ok, d, trash)

        def issue_idx(base):
            ics = [pltpu.async_copy(src_hbm.at[sid, base + s, 0],
                                    sbufs[s], iss[s]) for s in range(NBUF)]
            jcs = [pltpu.async_copy(dst_hbm.at[sid, base + s, 0],
                                    dbufs[s], jss[s]) for s in range(NBUF)]
            return ics, jcs

        def run_group(base, ics, jcs, wait_prev_scatter):
            gcs = []
            for s in range(NBUF):
                ics[s].wait()
                if wait_prev_scatter:
                    pltpu.make_async_copy(bufs[s], acc_sh.at[dls[s]],
                                          sss[s]).wait()
                gcs.append(pltpu.async_copy(g0_hbm.at[sbufs[s]],
                                            bufs[s], gss[s]))
            for s in range(NBUF):
                jcs[s].wait()
                remap(dbufs[s], dls[s])
            for s in range(NBUF):
                gcs[s].wait()
                pltpu.async_copy(bufs[s], acc_sh.at[dls[s]], sss[s],
                                 add=True)

        ics, jcs = issue_idx(0)
        run_group(0, ics, jcs, False)

        def group(g, _):
            base = NBUF * g
            ics, jcs = issue_idx(base)
            run_group(base, ics, jcs, True)
            return 0

        lax.fori_loop(1, C // NBUF, group, 0)
        for s in range(NBUF):
            pltpu.make_async_copy(bufs[s], acc_sh.at[dls[s]], sss[s]).wait()

        plsc.subcore_barrier()
        pltpu.sync_copy(
            acc_sh.at[pl.ds(sid * RPT, RPT)],
            out_hbm.at[cid, pl.ds(sid * RPT, RPT)])

        @pl.when(sid == 0)
        def _():
            pltpu.sync_copy(
                acc_sh.at[pl.ds(NS * RPT, REM)],
                out_hbm.at[cid, pl.ds(NS * RPT, REM)])

    return msg_kernel(g0, src_p, dst_p)


def _tc_matmul(x, W1p, kmask, deg2):
    N, IN = x.shape
    KP, H = W1p.shape
    RB = 2000
    KB = 1024
    NR = N // RB
    NK = KP // KB

    def body(x_ref, w_ref, km_ref, deg_ref, g0_ref, dinv_ref, acc_ref):
        k = pl.program_id(1)

        @pl.when(k == 0)
        def _():
            acc_ref[...] = jnp.zeros_like(acc_ref)

        @pl.when(k < NK - 1)
        def _():
            acc_ref[...] += jnp.dot(x_ref[...], w_ref[...],
                                    preferred_element_type=jnp.float32)

        @pl.when(k == NK - 1)
        def _():
            xb = x_ref[...] * km_ref[...]
            acc_ref[...] += jnp.dot(xb, w_ref[...],
                                    preferred_element_type=jnp.float32)

        @pl.when(k == NK - 1)
        def _():
            deg = deg_ref[:, 0] + deg_ref[:, 1] + 1.0
            dinv = lax.rsqrt(jnp.maximum(deg, 1e-12))
            dinv_ref[...] = dinv[:, None]
            g0_ref[...] = acc_ref[...] * dinv[:, None]

    return pl.pallas_call(
        body,
        grid=(NR, NK),
        in_specs=[
            pl.BlockSpec((RB, KB), lambda i, k: (i, k)),
            pl.BlockSpec((KB, H), lambda i, k: (k, 0)),
            pl.BlockSpec((1, KB), lambda i, k: (0, k)),
            pl.BlockSpec((RB, 2), lambda i, k: (i, 0)),
        ],
        out_specs=[
            pl.BlockSpec((RB, H), lambda i, k: (i, 0)),
            pl.BlockSpec((RB, 1), lambda i, k: (i, 0)),
        ],
        out_shape=[
            jax.ShapeDtypeStruct((N, H), jnp.float32),
            jax.ShapeDtypeStruct((N, 1), jnp.float32),
        ],
        scratch_shapes=[pltpu.VMEM((RB, H), jnp.float32)],
    )(x, W1p, kmask, deg2)


def _tc_epilogue(acc, g0, dinv, b1, cond_in, lm_W, lm_b,
                 Wg, bg, Wb, bb, Wp, bp, Wo, bo):
    N, H = g0.shape
    Bc, CIN = cond_in.shape
    CH = lm_W.shape[1]
    O = Wo.shape[1]
    RB = 2000
    NR = N // RB

    def body(acc_ref, g0_ref, dinv_ref, b1_ref, cond_ref,
             lmW_ref, lmb_ref, Wg_ref, bg_ref, Wb_ref, bb_ref,
             Wp_ref, bp_ref, Wo_ref, bo_ref, out_ref, M_acc):
        i = pl.program_id(0)

        @pl.when(i == 0)
        def _():
            M_acc[...] = jnp.zeros_like(M_acc)

        dv = dinv_ref[:, 0]
        hb = jnp.maximum(
            dv[:, None] * (acc_ref[...] + g0_ref[...]) + b1_ref[0, :], 0.0)
        M_acc[...] += lax.dot_general(
            hb, Wp_ref[...], (((0,), (0,)), ((), ())),
            preferred_element_type=jnp.float32)

        @pl.when(i == NR - 1)
        def _():
            cond = jnp.dot(cond_ref[...], lmW_ref[...],
                           preferred_element_type=jnp.float32) + lmb_ref[0, :]
            gamma = jnp.dot(cond, Wg_ref[...],
                            preferred_element_type=jnp.float32) + bg_ref[0, :]
            beta = jnp.dot(cond, Wb_ref[...],
                           preferred_element_type=jnp.float32) + bb_ref[0, :]
            z = jnp.dot(gamma, M_acc[...],
                        preferred_element_type=jnp.float32) + bp_ref[0, :]
            z = jnp.maximum(z + beta, 0.0)
            o = jnp.dot(z, Wo_ref[...],
                        preferred_element_type=jnp.float32) + bo_ref[0, :]
            m = jnp.max(o, axis=1, keepdims=True)
            lse = m + jnp.log(jnp.sum(jnp.exp(o - m), axis=1, keepdims=True))
            out_ref[...] = o - lse

    full = lambda shape: pl.BlockSpec(shape, lambda i: tuple(0 for _ in shape))
    row = lambda w: pl.BlockSpec((RB, w), lambda i: (i, 0))
    return pl.pallas_call(
        body,
        grid=(NR,),
        in_specs=[
            row(H),
            row(H),
            row(1),
            full((1, H)),
            full((Bc, CIN)),
            full((CIN, CH)),
            full((1, CH)),
            full((CH, H)),
            full((1, H)),
            full((CH, H)),
            full((1, H)),
            row(H),
            full((1, H)),
            full((H, O)),
            full((1, O)),
        ],
        out_specs=pl.BlockSpec((Bc, O), lambda i: (0, 0)),
        out_shape=jax.ShapeDtypeStruct((Bc, O), jnp.float32),
        scratch_shapes=[pltpu.VMEM((H, H), jnp.float32)],
    )(acc, g0, dinv, b1, cond_in, lm_W, lm_b,
      Wg, bg, Wb, bb, Wp, bp, Wo, bo)


def kernel(x, edge_index, condition, hidden_dim, lm_W, lm_b, Wg, bg, Wb, bb,
           W1, b1, Wp, bp, Wo, bo):
    del hidden_dim
    N, IN = x.shape
    H = W1.shape[1]
    E = edge_index.shape[1]

    info = plsc.get_sparse_core_info()
    NC, NS = info.num_cores, info.num_subcores
    ET = E // NS
    K_CH = 80
    C_CH = ET // K_CH

    src_p = edge_index[0].reshape(NS, C_CH, 1, K_CH)
    dst_p = edge_index[1].reshape(NS, C_CH, 1, K_CH)

    KB = 1024
    KP = ((IN + KB - 1) // KB) * KB
    W1p = jnp.zeros((KP, H), jnp.float32).at[:IN].set(W1)
    kmask = (jnp.arange(KP, dtype=jnp.int32) < IN).astype(jnp.float32)[None, :]

    deg2 = _sc_degree(dst_p, N, NC, NS)
    g0, dinv = _tc_matmul(x, W1p, kmask, deg2.reshape(NC, N).T)
    accp = _sc_message(g0, src_p, dst_p, NC, NS)

    return _tc_epilogue(
        accp.reshape(N, H), g0, dinv,
        b1.reshape(1, H), condition, lm_W, lm_b.reshape(1, -1),
        Wg, bg.reshape(1, H), Wb, bb.reshape(1, H),
        Wp, bp.reshape(1, H), Wo, bo.reshape(1, -1))
